# baseline (speedup 1.0000x reference)
import functools

import jax
import jax.numpy as jnp
from jax import lax
from jax.experimental import pallas as pl
from jax.experimental.pallas import tpu as pltpu
from jax.experimental.pallas import tpu_sc as plsc

_B, _S, _H = 64, 512, 768
_BS = _B * _S
_EPS = 1e-12
_CH = 64


def _sc_info():
    info = plsc.get_sparse_core_info()
    return info.num_cores, info.num_subcores


def _sc_gather_body(nc, tpw, nchunk, ids_hbm, word_hbm, out_hbm,
                    idx0, idx1, buf0, buf1, gsem0, gsem1, osem0, osem1):
    wid = lax.axis_index("s") * nc + lax.axis_index("c")
    base = wid * tpw
    idxs = (idx0, idx1)
    bufs = (buf0, buf1)
    gsems = (gsem0, gsem1)
    osems = (osem0, osem1)

    def stage_and_fire(c):
        k = c & 1
        tok0 = base + c * _CH
        pltpu.sync_copy(ids_hbm.at[pl.ds(tok0, _CH)], idxs[k])
        return pltpu.async_copy(word_hbm.at[idxs[k]], bufs[k], gsems[k])

    gather = {0: stage_and_fire(0)}
    store = {}
    for c in range(nchunk):
        k = c & 1
        if c + 1 < nchunk:
            if c - 1 >= 0:
                store[c - 1].wait()
            gather[c + 1] = stage_and_fire(c + 1)
        gather[c].wait()
        tok0 = base + c * _CH
        store[c] = pltpu.async_copy(bufs[k], out_hbm.at[pl.ds(tok0, _CH)],
                                    osems[k])
    store[nchunk - 2].wait()
    store[nchunk - 1].wait()


def _make_sc_gather():
    nc, ns = _sc_info()
    nw = nc * ns
    tpw = _BS // nw
    mesh = plsc.VectorSubcoreMesh(core_axis_name="c", subcore_axis_name="s")
    return pl.kernel(
        functools.partial(_sc_gather_body, nc, tpw, tpw // _CH),
        mesh=mesh,
        out_type=jax.ShapeDtypeStruct((_BS, _H), jnp.float32),
        scratch_types=[
            pltpu.VMEM((_CH,), jnp.int32),
            pltpu.VMEM((_CH,), jnp.int32),
            pltpu.VMEM((_CH, _H), jnp.float32),
            pltpu.VMEM((_CH, _H), jnp.float32),
            pltpu.SemaphoreType.DMA,
            pltpu.SemaphoreType.DMA,
            pltpu.SemaphoreType.DMA,
            pltpu.SemaphoreType.DMA,
        ],
    )


_SB = 128


def _tc_ln_body(x_ref, pos_ref, tt_ref, t0_ref, d_ref, g_ref, b_ref, o_ref):
    ttf = tt_ref[0].astype(jnp.float32)
    ttcol = jnp.transpose(ttf)
    x = x_ref[0] + pos_ref[...] + t0_ref[0:1] + ttcol * d_ref[0:1]
    mean = jnp.mean(x, axis=-1, keepdims=True)
    msq = jnp.mean(x * x, axis=-1, keepdims=True)
    inv = lax.rsqrt(msq - mean * mean + _EPS)
    o_ref[0] = (x - mean) * inv * g_ref[0:1] + b_ref[0:1]


def _tc_ln(x, pos_table, tt3, t08, d8, gamma8, beta8):
    return pl.pallas_call(
        _tc_ln_body,
        grid=(_S // _SB, _B),
        in_specs=[
            pl.BlockSpec((1, _SB, _H), lambda i, j: (j, i, 0)),
            pl.BlockSpec((_SB, _H), lambda i, j: (i, 0)),
            pl.BlockSpec((1, 1, _SB), lambda i, j: (j, 0, i)),
            pl.BlockSpec((8, _H), lambda i, j: (0, 0)),
            pl.BlockSpec((8, _H), lambda i, j: (0, 0)),
            pl.BlockSpec((8, _H), lambda i, j: (0, 0)),
            pl.BlockSpec((8, _H), lambda i, j: (0, 0)),
        ],
        out_specs=pl.BlockSpec((1, _SB, _H), lambda i, j: (j, i, 0)),
        out_shape=jax.ShapeDtypeStruct((_B, _S, _H), jnp.float32),
    )(x, pos_table, tt3, t08, d8, gamma8, beta8)


def kernel(input_ids, token_type_ids, word_table, pos_table, type_table,
           ln_gamma, ln_beta):
    ids = input_ids.reshape(-1).astype(jnp.int32)
    tt3 = token_type_ids.reshape(_B, 1, _S).astype(jnp.int32)
    sc_gather = _make_sc_gather()
    words = sc_gather(ids, word_table)
    t0 = type_table[0]
    d = type_table[1] - t0
    t08 = jnp.broadcast_to(t0[None, :], (8, _H))
    d8 = jnp.broadcast_to(d[None, :], (8, _H))
    gamma8 = jnp.broadcast_to(ln_gamma[None, :], (8, _H))
    beta8 = jnp.broadcast_to(ln_beta[None, :], (8, _H))
    out = _tc_ln(words.reshape(_B, _S, _H), pos_table, tt3, t08, d8,
                 gamma8, beta8)
    mask = jnp.ones((_B, _S), dtype=jnp.int32)
    return (out, mask)


# speedup vs baseline: 1.5275x; 1.5275x over previous
import functools

import jax
import jax.numpy as jnp
from jax import lax
from jax.experimental import pallas as pl
from jax.experimental.pallas import tpu as pltpu
from jax.experimental.pallas import tpu_sc as plsc

_B, _S, _H = 64, 512, 768
_BS = _B * _S
_EPS = 1e-12
_CH = 64


def _sc_info():
    info = plsc.get_sparse_core_info()
    return info.num_cores, info.num_subcores


def _sc_gather_body(nc, tpw, nchunk, ids_hbm, word_hbm, out_hbm,
                    idx0, idx1, buf0, buf1, gsem0, gsem1, osem0, osem1):
    wid = lax.axis_index("s") * nc + lax.axis_index("c")
    base = wid * tpw
    idxs = (idx0, idx1)
    bufs = (buf0, buf1)
    gsems = (gsem0, gsem1)
    osems = (osem0, osem1)

    def stage_and_fire(c):
        k = c & 1
        tok0 = base + c * _CH
        pltpu.sync_copy(ids_hbm.at[pl.ds(tok0, _CH)], idxs[k])
        return pltpu.async_copy(word_hbm.at[idxs[k]], bufs[k], gsems[k])

    gather = {0: stage_and_fire(0)}
    store = {}
    for c in range(nchunk):
        k = c & 1
        if c + 1 < nchunk:
            if c - 1 >= 0:
                store[c - 1].wait()
            gather[c + 1] = stage_and_fire(c + 1)
        gather[c].wait()
        tok0 = base + c * _CH
        store[c] = pltpu.async_copy(bufs[k], out_hbm.at[pl.ds(tok0, _CH)],
                                    osems[k])
    store[nchunk - 2].wait()
    store[nchunk - 1].wait()


def _make_sc_gather():
    nc, ns = _sc_info()
    nw = nc * ns
    tpw = _BS // nw
    mesh = plsc.VectorSubcoreMesh(core_axis_name="c", subcore_axis_name="s")
    return pl.kernel(
        functools.partial(_sc_gather_body, nc, tpw, tpw // _CH),
        mesh=mesh,
        out_type=jax.ShapeDtypeStruct((_BS, _H), jnp.float32),
        scratch_types=[
            pltpu.VMEM((_CH,), jnp.int32),
            pltpu.VMEM((_CH,), jnp.int32),
            pltpu.VMEM((_CH, _H), jnp.float32),
            pltpu.VMEM((_CH, _H), jnp.float32),
            pltpu.SemaphoreType.DMA,
            pltpu.SemaphoreType.DMA,
            pltpu.SemaphoreType.DMA,
            pltpu.SemaphoreType.DMA,
        ],
    )


def _tc_ln_body(x_ref, pos_ref, tt_ref, t0_ref, d_ref, g_ref, b_ref, o_ref):
    ttf = tt_ref[0].astype(jnp.float32)
    ttcol = jnp.transpose(ttf)
    x = x_ref[0] + pos_ref[...] + t0_ref[0:1] + ttcol * d_ref[0:1]
    mean = jnp.mean(x, axis=-1, keepdims=True)
    msq = jnp.mean(x * x, axis=-1, keepdims=True)
    inv = lax.rsqrt(msq - mean * mean + _EPS)
    o_ref[0] = (x - mean) * inv * g_ref[0:1] + b_ref[0:1]


def _tc_ln(x, pos_table, tt3, t08, d8, gamma8, beta8):
    return pl.pallas_call(
        _tc_ln_body,
        grid=(_B,),
        in_specs=[
            pl.BlockSpec((1, _S, _H), lambda i: (i, 0, 0)),
            pl.BlockSpec((_S, _H), lambda i: (0, 0)),
            pl.BlockSpec((1, 1, _S), lambda i: (i, 0, 0)),
            pl.BlockSpec((8, _H), lambda i: (0, 0)),
            pl.BlockSpec((8, _H), lambda i: (0, 0)),
            pl.BlockSpec((8, _H), lambda i: (0, 0)),
            pl.BlockSpec((8, _H), lambda i: (0, 0)),
        ],
        out_specs=pl.BlockSpec((1, _S, _H), lambda i: (i, 0, 0)),
        out_shape=jax.ShapeDtypeStruct((_B, _S, _H), jnp.float32),
    )(x, pos_table, tt3, t08, d8, gamma8, beta8)


def kernel(input_ids, token_type_ids, word_table, pos_table, type_table,
           ln_gamma, ln_beta):
    ids = input_ids.reshape(-1).astype(jnp.int32)
    tt3 = token_type_ids.reshape(_B, 1, _S).astype(jnp.int32)
    sc_gather = _make_sc_gather()
    words = sc_gather(ids, word_table)
    t0 = type_table[0]
    d = type_table[1] - t0
    t08 = jnp.broadcast_to(t0[None, :], (8, _H))
    d8 = jnp.broadcast_to(d[None, :], (8, _H))
    gamma8 = jnp.broadcast_to(ln_gamma[None, :], (8, _H))
    beta8 = jnp.broadcast_to(ln_beta[None, :], (8, _H))
    out = _tc_ln(words.reshape(_B, _S, _H), pos_table, tt3, t08, d8,
                 gamma8, beta8)
    mask = jnp.ones((_B, _S), dtype=jnp.int32)
    return (out, mask)


# speedup vs baseline: 1.6135x; 1.0563x over previous
import functools

import jax
import jax.numpy as jnp
from jax import lax
from jax.experimental import pallas as pl
from jax.experimental.pallas import tpu as pltpu
from jax.experimental.pallas import tpu_sc as plsc

_B, _S, _H = 64, 512, 768
_BS = _B * _S
_EPS = 1e-12
_CH = 64
_NSPLIT = 2


def _sc_info():
    info = plsc.get_sparse_core_info()
    return info.num_cores, info.num_subcores


def _sc_gather_body(nc, tpw, nchunk, ids_hbm, word_hbm, out_hbm,
                    idx0, idx1, buf0, buf1, gsem0, gsem1, osem0, osem1):
    wid = lax.axis_index("s") * nc + lax.axis_index("c")
    base = wid * tpw
    idxs = (idx0, idx1)
    bufs = (buf0, buf1)
    gsems = (gsem0, gsem1)
    osems = (osem0, osem1)

    def stage_and_fire(c):
        k = c & 1
        tok0 = base + c * _CH
        pltpu.sync_copy(ids_hbm.at[pl.ds(tok0, _CH)], idxs[k])
        return pltpu.async_copy(word_hbm.at[idxs[k]], bufs[k], gsems[k])

    gather = {0: stage_and_fire(0)}
    store = {}
    for c in range(nchunk):
        k = c & 1
        if c + 1 < nchunk:
            if c - 1 >= 0:
                store[c - 1].wait()
            gather[c + 1] = stage_and_fire(c + 1)
        gather[c].wait()
        tok0 = base + c * _CH
        store[c] = pltpu.async_copy(bufs[k], out_hbm.at[pl.ds(tok0, _CH)],
                                    osems[k])
    store[nchunk - 2].wait()
    store[nchunk - 1].wait()


def _make_sc_gather(ntok):
    nc, ns = _sc_info()
    nw = nc * ns
    tpw = ntok // nw
    mesh = plsc.VectorSubcoreMesh(core_axis_name="c", subcore_axis_name="s")
    return pl.kernel(
        functools.partial(_sc_gather_body, nc, tpw, tpw // _CH),
        mesh=mesh,
        out_type=jax.ShapeDtypeStruct((ntok, _H), jnp.float32),
        scratch_types=[
            pltpu.VMEM((_CH,), jnp.int32),
            pltpu.VMEM((_CH,), jnp.int32),
            pltpu.VMEM((_CH, _H), jnp.float32),
            pltpu.VMEM((_CH, _H), jnp.float32),
            pltpu.SemaphoreType.DMA,
            pltpu.SemaphoreType.DMA,
            pltpu.SemaphoreType.DMA,
            pltpu.SemaphoreType.DMA,
        ],
    )


def _tc_ln_body(x_ref, pos_ref, tt_ref, t0_ref, d_ref, g_ref, b_ref, o_ref):
    ttf = tt_ref[0].astype(jnp.float32)
    ttcol = jnp.transpose(ttf)
    x = x_ref[0] + pos_ref[...] + t0_ref[0:1] + ttcol * d_ref[0:1]
    mean = jnp.mean(x, axis=-1, keepdims=True)
    msq = jnp.mean(x * x, axis=-1, keepdims=True)
    inv = lax.rsqrt(msq - mean * mean + _EPS)
    o_ref[0] = (x - mean) * inv * g_ref[0:1] + b_ref[0:1]


def _tc_ln_part_body(x_ref, pos_ref, tt_ref, t0_ref, d_ref, g_ref, b_ref,
                     dest_ref, o_ref):
    del dest_ref
    _tc_ln_body(x_ref, pos_ref, tt_ref, t0_ref, d_ref, g_ref, b_ref, o_ref)


def _tc_ln_part(x, pos_table, tt3, t08, d8, gamma8, beta8, off, dest):
    nb = x.shape[0]
    base_specs = [
        pl.BlockSpec((1, _S, _H), lambda i: (i, 0, 0)),
        pl.BlockSpec((_S, _H), lambda i: (0, 0)),
        pl.BlockSpec((1, 1, _S), lambda i: (i, 0, 0)),
        pl.BlockSpec((8, _H), lambda i: (0, 0)),
        pl.BlockSpec((8, _H), lambda i: (0, 0)),
        pl.BlockSpec((8, _H), lambda i: (0, 0)),
        pl.BlockSpec((8, _H), lambda i: (0, 0)),
    ]
    out_spec = pl.BlockSpec((1, _S, _H), lambda i, off=off: (i + off, 0, 0))
    out_shape = jax.ShapeDtypeStruct((_B, _S, _H), jnp.float32)
    if dest is None:
        return pl.pallas_call(
            _tc_ln_body,
            grid=(nb,),
            in_specs=base_specs,
            out_specs=out_spec,
            out_shape=out_shape,
        )(x, pos_table, tt3, t08, d8, gamma8, beta8)
    return pl.pallas_call(
        _tc_ln_part_body,
        grid=(nb,),
        in_specs=base_specs + [pl.BlockSpec(memory_space=pl.ANY)],
        out_specs=out_spec,
        out_shape=out_shape,
        input_output_aliases={7: 0},
    )(x, pos_table, tt3, t08, d8, gamma8, beta8, dest)


def kernel(input_ids, token_type_ids, word_table, pos_table, type_table,
           ln_gamma, ln_beta):
    ids = input_ids.reshape(-1).astype(jnp.int32)
    tt3 = token_type_ids.reshape(_B, 1, _S).astype(jnp.int32)
    t0 = type_table[0]
    d = type_table[1] - t0
    t08 = jnp.broadcast_to(t0[None, :], (8, _H))
    d8 = jnp.broadcast_to(d[None, :], (8, _H))
    gamma8 = jnp.broadcast_to(ln_gamma[None, :], (8, _H))
    beta8 = jnp.broadcast_to(ln_beta[None, :], (8, _H))

    nbp = _B // _NSPLIT
    ntok = nbp * _S
    sc_gather = _make_sc_gather(ntok)
    words = [sc_gather(lax.dynamic_slice_in_dim(ids, k * ntok, ntok),
                       word_table)
             for k in range(_NSPLIT)]
    out = None
    for k in range(_NSPLIT):
        out = _tc_ln_part(
            words[k].reshape(nbp, _S, _H), pos_table,
            lax.dynamic_slice_in_dim(tt3, k * nbp, nbp),
            t08, d8, gamma8, beta8, off=k * nbp, dest=out)
    mask = jnp.ones((_B, _S), dtype=jnp.int32)
    return (out, mask)


# speedup vs baseline: 1.6364x; 1.0142x over previous
import functools

import jax
import jax.numpy as jnp
from jax import lax
from jax.experimental import pallas as pl
from jax.experimental.pallas import tpu as pltpu
from jax.experimental.pallas import tpu_sc as plsc

_B, _S, _H = 64, 512, 768
_BS = _B * _S
_EPS = 1e-12
_CH = 64
_NSPLIT = 4


def _sc_info():
    info = plsc.get_sparse_core_info()
    return info.num_cores, info.num_subcores


def _sc_gather_body(nc, tpw, nchunk, ids_hbm, word_hbm, out_hbm,
                    idx0, idx1, buf0, buf1, gsem0, gsem1, osem0, osem1):
    wid = lax.axis_index("s") * nc + lax.axis_index("c")
    base = wid * tpw
    idxs = (idx0, idx1)
    bufs = (buf0, buf1)
    gsems = (gsem0, gsem1)
    osems = (osem0, osem1)

    def stage_and_fire(c):
        k = c & 1
        tok0 = base + c * _CH
        pltpu.sync_copy(ids_hbm.at[pl.ds(tok0, _CH)], idxs[k])
        return pltpu.async_copy(word_hbm.at[idxs[k]], bufs[k], gsems[k])

    gather = {0: stage_and_fire(0)}
    store = {}
    for c in range(nchunk):
        k = c & 1
        if c + 1 < nchunk:
            if c - 1 >= 0:
                store[c - 1].wait()
            gather[c + 1] = stage_and_fire(c + 1)
        gather[c].wait()
        tok0 = base + c * _CH
        store[c] = pltpu.async_copy(bufs[k], out_hbm.at[pl.ds(tok0, _CH)],
                                    osems[k])
    store[nchunk - 2].wait()
    store[nchunk - 1].wait()


def _make_sc_gather(ntok):
    nc, ns = _sc_info()
    nw = nc * ns
    tpw = ntok // nw
    mesh = plsc.VectorSubcoreMesh(core_axis_name="c", subcore_axis_name="s")
    return pl.kernel(
        functools.partial(_sc_gather_body, nc, tpw, tpw // _CH),
        mesh=mesh,
        out_type=jax.ShapeDtypeStruct((ntok, _H), jnp.float32),
        scratch_types=[
            pltpu.VMEM((_CH,), jnp.int32),
            pltpu.VMEM((_CH,), jnp.int32),
            pltpu.VMEM((_CH, _H), jnp.float32),
            pltpu.VMEM((_CH, _H), jnp.float32),
            pltpu.SemaphoreType.DMA,
            pltpu.SemaphoreType.DMA,
            pltpu.SemaphoreType.DMA,
            pltpu.SemaphoreType.DMA,
        ],
    )


def _tc_ln_body(x_ref, pos_ref, tt_ref, t0_ref, d_ref, g_ref, b_ref, o_ref):
    ttf = tt_ref[0].astype(jnp.float32)
    ttcol = jnp.transpose(ttf)
    x = x_ref[0] + pos_ref[...] + t0_ref[0:1] + ttcol * d_ref[0:1]
    mean = jnp.mean(x, axis=-1, keepdims=True)
    msq = jnp.mean(x * x, axis=-1, keepdims=True)
    inv = lax.rsqrt(msq - mean * mean + _EPS)
    o_ref[0] = (x - mean) * inv * g_ref[0:1] + b_ref[0:1]


def _tc_ln_part_body(x_ref, pos_ref, tt_ref, t0_ref, d_ref, g_ref, b_ref,
                     dest_ref, o_ref):
    del dest_ref
    _tc_ln_body(x_ref, pos_ref, tt_ref, t0_ref, d_ref, g_ref, b_ref, o_ref)


def _tc_ln_part(x, pos_table, tt3, t08, d8, gamma8, beta8, off, dest):
    nb = x.shape[0]
    base_specs = [
        pl.BlockSpec((1, _S, _H), lambda i: (i, 0, 0)),
        pl.BlockSpec((_S, _H), lambda i: (0, 0)),
        pl.BlockSpec((1, 1, _S), lambda i: (i, 0, 0)),
        pl.BlockSpec((8, _H), lambda i: (0, 0)),
        pl.BlockSpec((8, _H), lambda i: (0, 0)),
        pl.BlockSpec((8, _H), lambda i: (0, 0)),
        pl.BlockSpec((8, _H), lambda i: (0, 0)),
    ]
    out_spec = pl.BlockSpec((1, _S, _H), lambda i, off=off: (i + off, 0, 0))
    out_shape = jax.ShapeDtypeStruct((_B, _S, _H), jnp.float32)
    if dest is None:
        return pl.pallas_call(
            _tc_ln_body,
            grid=(nb,),
            in_specs=base_specs,
            out_specs=out_spec,
            out_shape=out_shape,
        )(x, pos_table, tt3, t08, d8, gamma8, beta8)
    return pl.pallas_call(
        _tc_ln_part_body,
        grid=(nb,),
        in_specs=base_specs + [pl.BlockSpec(memory_space=pl.ANY)],
        out_specs=out_spec,
        out_shape=out_shape,
        input_output_aliases={7: 0},
    )(x, pos_table, tt3, t08, d8, gamma8, beta8, dest)


def kernel(input_ids, token_type_ids, word_table, pos_table, type_table,
           ln_gamma, ln_beta):
    ids = input_ids.reshape(-1).astype(jnp.int32)
    tt3 = token_type_ids.reshape(_B, 1, _S).astype(jnp.int32)
    t0 = type_table[0]
    d = type_table[1] - t0
    t08 = jnp.broadcast_to(t0[None, :], (8, _H))
    d8 = jnp.broadcast_to(d[None, :], (8, _H))
    gamma8 = jnp.broadcast_to(ln_gamma[None, :], (8, _H))
    beta8 = jnp.broadcast_to(ln_beta[None, :], (8, _H))

    nbp = _B // _NSPLIT
    ntok = nbp * _S
    sc_gather = _make_sc_gather(ntok)
    words = [sc_gather(lax.dynamic_slice_in_dim(ids, k * ntok, ntok),
                       word_table)
             for k in range(_NSPLIT)]
    out = None
    for k in range(_NSPLIT):
        out = _tc_ln_part(
            words[k].reshape(nbp, _S, _H), pos_table,
            lax.dynamic_slice_in_dim(tt3, k * nbp, nbp),
            t08, d8, gamma8, beta8, off=k * nbp, dest=out)
    mask = jnp.ones((_B, _S), dtype=jnp.int32)
    return (out, mask)


# speedup vs baseline: 1.7288x; 1.0565x over previous
import functools

import jax
import jax.numpy as jnp
from jax import lax
from jax.experimental import pallas as pl
from jax.experimental.pallas import tpu as pltpu
from jax.experimental.pallas import tpu_sc as plsc

_B, _S, _H = 64, 512, 768
_BS = _B * _S
_EPS = 1e-12
_CH = 64
_NSPLIT = 4


def _sc_info():
    info = plsc.get_sparse_core_info()
    return info.num_cores, info.num_subcores


def _sc_gather_body(nc, tpw, nchunk, ids_hbm, word_hbm, out_hbm,
                    idx0, idx1, buf0, buf1, gsem0, gsem1, osem0, osem1):
    wid = lax.axis_index("s") * nc + lax.axis_index("c")
    base = wid * tpw
    idxs = (idx0, idx1)
    bufs = (buf0, buf1)
    gsems = (gsem0, gsem1)
    osems = (osem0, osem1)

    def stage_and_fire(c):
        k = c & 1
        tok0 = base + c * _CH
        pltpu.sync_copy(ids_hbm.at[pl.ds(tok0, _CH)], idxs[k])
        return pltpu.async_copy(word_hbm.at[idxs[k]], bufs[k], gsems[k])

    gather = {0: stage_and_fire(0)}
    store = {}
    for c in range(nchunk):
        k = c & 1
        if c + 1 < nchunk:
            if c - 1 >= 0:
                store[c - 1].wait()
            gather[c + 1] = stage_and_fire(c + 1)
        gather[c].wait()
        tok0 = base + c * _CH
        store[c] = pltpu.async_copy(bufs[k], out_hbm.at[pl.ds(tok0, _CH)],
                                    osems[k])
    store[nchunk - 2].wait()
    store[nchunk - 1].wait()


def _make_sc_gather(ntok):
    nc, ns = _sc_info()
    nw = nc * ns
    tpw = ntok // nw
    mesh = plsc.VectorSubcoreMesh(core_axis_name="c", subcore_axis_name="s")
    return pl.kernel(
        functools.partial(_sc_gather_body, nc, tpw, tpw // _CH),
        mesh=mesh,
        out_type=jax.ShapeDtypeStruct((ntok, _H), jnp.float32),
        scratch_types=[
            pltpu.VMEM((_CH,), jnp.int32),
            pltpu.VMEM((_CH,), jnp.int32),
            pltpu.VMEM((_CH, _H), jnp.float32),
            pltpu.VMEM((_CH, _H), jnp.float32),
            pltpu.SemaphoreType.DMA,
            pltpu.SemaphoreType.DMA,
            pltpu.SemaphoreType.DMA,
            pltpu.SemaphoreType.DMA,
        ],
    )


_NBB = 2


def _tc_ln_body(x_ref, pos_ref, tt_ref, t0_ref, d_ref, g_ref, b_ref, o_ref):
    ttf = tt_ref[...].astype(jnp.float32)
    ttcol = jnp.swapaxes(ttf, 1, 2)
    x = x_ref[...] + (pos_ref[...] + t0_ref[0:1])[None] + ttcol * d_ref[0:1][None]
    mean = jnp.mean(x, axis=-1, keepdims=True)
    msq = jnp.mean(x * x, axis=-1, keepdims=True)
    inv = lax.rsqrt(msq - mean * mean + _EPS)
    o_ref[...] = (x - mean) * inv * g_ref[0:1][None] + b_ref[0:1][None]


def _tc_ln_part_body(x_ref, pos_ref, tt_ref, t0_ref, d_ref, g_ref, b_ref,
                     dest_ref, o_ref):
    del dest_ref
    _tc_ln_body(x_ref, pos_ref, tt_ref, t0_ref, d_ref, g_ref, b_ref, o_ref)


def _tc_ln_part(x, pos_table, tt3, t08, d8, gamma8, beta8, off, dest):
    nb = x.shape[0] // _NBB
    offb = off // _NBB
    base_specs = [
        pl.BlockSpec((_NBB, _S, _H), lambda i: (i, 0, 0)),
        pl.BlockSpec((_S, _H), lambda i: (0, 0)),
        pl.BlockSpec((_NBB, 1, _S), lambda i: (i, 0, 0)),
        pl.BlockSpec((8, _H), lambda i: (0, 0)),
        pl.BlockSpec((8, _H), lambda i: (0, 0)),
        pl.BlockSpec((8, _H), lambda i: (0, 0)),
        pl.BlockSpec((8, _H), lambda i: (0, 0)),
    ]
    out_spec = pl.BlockSpec((_NBB, _S, _H),
                            lambda i, offb=offb: (i + offb, 0, 0))
    out_shape = jax.ShapeDtypeStruct((_B, _S, _H), jnp.float32)
    if dest is None:
        return pl.pallas_call(
            _tc_ln_body,
            grid=(nb,),
            in_specs=base_specs,
            out_specs=out_spec,
            out_shape=out_shape,
        )(x, pos_table, tt3, t08, d8, gamma8, beta8)
    return pl.pallas_call(
        _tc_ln_part_body,
        grid=(nb,),
        in_specs=base_specs + [pl.BlockSpec(memory_space=pl.ANY)],
        out_specs=out_spec,
        out_shape=out_shape,
        input_output_aliases={7: 0},
    )(x, pos_table, tt3, t08, d8, gamma8, beta8, dest)


def kernel(input_ids, token_type_ids, word_table, pos_table, type_table,
           ln_gamma, ln_beta):
    ids = input_ids.reshape(-1).astype(jnp.int32)
    tt3 = token_type_ids.reshape(_B, 1, _S).astype(jnp.int32)
    t0 = type_table[0]
    d = type_table[1] - t0
    t08 = jnp.broadcast_to(t0[None, :], (8, _H))
    d8 = jnp.broadcast_to(d[None, :], (8, _H))
    gamma8 = jnp.broadcast_to(ln_gamma[None, :], (8, _H))
    beta8 = jnp.broadcast_to(ln_beta[None, :], (8, _H))

    nbp = _B // _NSPLIT
    ntok = nbp * _S
    sc_gather = _make_sc_gather(ntok)
    words = [sc_gather(lax.dynamic_slice_in_dim(ids, k * ntok, ntok),
                       word_table)
             for k in range(_NSPLIT)]
    out = None
    for k in range(_NSPLIT):
        out = _tc_ln_part(
            words[k].reshape(nbp, _S, _H), pos_table,
            lax.dynamic_slice_in_dim(tt3, k * nbp, nbp),
            t08, d8, gamma8, beta8, off=k * nbp, dest=out)
    mask = jnp.ones((_B, _S), dtype=jnp.int32)
    return (out, mask)


# speedup vs baseline: 1.7798x; 1.0295x over previous
import functools

import jax
import jax.numpy as jnp
from jax import lax
from jax.experimental import pallas as pl
from jax.experimental.pallas import tpu as pltpu
from jax.experimental.pallas import tpu_sc as plsc

_B, _S, _H = 64, 512, 768
_BS = _B * _S
_EPS = 1e-12
_CH = 64
_NSPLIT = 4


def _sc_info():
    info = plsc.get_sparse_core_info()
    return info.num_cores, info.num_subcores


def _sc_gather_body(nc, tpw, nchunk, ids_hbm, word_hbm, out_hbm,
                    idx0, idx1, buf0, buf1, gsem0, gsem1, osem0, osem1):
    wid = lax.axis_index("s") * nc + lax.axis_index("c")
    base = wid * tpw
    idxs = (idx0, idx1)
    bufs = (buf0, buf1)
    gsems = (gsem0, gsem1)
    osems = (osem0, osem1)

    def stage_and_fire(c):
        k = c & 1
        tok0 = base + c * _CH
        pltpu.sync_copy(ids_hbm.at[pl.ds(tok0, _CH)], idxs[k])
        return pltpu.async_copy(word_hbm.at[idxs[k]], bufs[k], gsems[k])

    gather = {0: stage_and_fire(0)}
    store = {}
    for c in range(nchunk):
        k = c & 1
        if c + 1 < nchunk:
            if c - 1 >= 0:
                store[c - 1].wait()
            gather[c + 1] = stage_and_fire(c + 1)
        gather[c].wait()
        tok0 = base + c * _CH
        store[c] = pltpu.async_copy(bufs[k], out_hbm.at[pl.ds(tok0, _CH)],
                                    osems[k])
    store[nchunk - 2].wait()
    store[nchunk - 1].wait()


def _make_sc_gather(ntok):
    nc, ns = _sc_info()
    nw = nc * ns
    tpw = ntok // nw
    mesh = plsc.VectorSubcoreMesh(core_axis_name="c", subcore_axis_name="s")
    return pl.kernel(
        functools.partial(_sc_gather_body, nc, tpw, tpw // _CH),
        mesh=mesh,
        out_type=jax.ShapeDtypeStruct((ntok, _H), jnp.float32),
        scratch_types=[
            pltpu.VMEM((_CH,), jnp.int32),
            pltpu.VMEM((_CH,), jnp.int32),
            pltpu.VMEM((_CH, _H), jnp.float32),
            pltpu.VMEM((_CH, _H), jnp.float32),
            pltpu.SemaphoreType.DMA,
            pltpu.SemaphoreType.DMA,
            pltpu.SemaphoreType.DMA,
            pltpu.SemaphoreType.DMA,
        ],
    )


_NBB = 4


def _tc_ln_body(x_ref, pos_ref, tt_ref, t0_ref, d_ref, g_ref, b_ref, o_ref):
    ttf = tt_ref[...].astype(jnp.float32)
    ttcol = jnp.swapaxes(ttf, 1, 2)
    x = x_ref[...] + (pos_ref[...] + t0_ref[0:1])[None] + ttcol * d_ref[0:1][None]
    mean = jnp.mean(x, axis=-1, keepdims=True)
    msq = jnp.mean(x * x, axis=-1, keepdims=True)
    inv = lax.rsqrt(msq - mean * mean + _EPS)
    o_ref[...] = (x - mean) * inv * g_ref[0:1][None] + b_ref[0:1][None]


def _tc_ln_part_body(x_ref, pos_ref, tt_ref, t0_ref, d_ref, g_ref, b_ref,
                     dest_ref, o_ref):
    del dest_ref
    _tc_ln_body(x_ref, pos_ref, tt_ref, t0_ref, d_ref, g_ref, b_ref, o_ref)


def _tc_ln_part(x, pos_table, tt3, t08, d8, gamma8, beta8, off, dest):
    nb = x.shape[0] // _NBB
    offb = off // _NBB
    base_specs = [
        pl.BlockSpec((_NBB, _S, _H), lambda i: (i, 0, 0)),
        pl.BlockSpec((_S, _H), lambda i: (0, 0)),
        pl.BlockSpec((_NBB, 1, _S), lambda i: (i, 0, 0)),
        pl.BlockSpec((8, _H), lambda i: (0, 0)),
        pl.BlockSpec((8, _H), lambda i: (0, 0)),
        pl.BlockSpec((8, _H), lambda i: (0, 0)),
        pl.BlockSpec((8, _H), lambda i: (0, 0)),
    ]
    out_spec = pl.BlockSpec((_NBB, _S, _H),
                            lambda i, offb=offb: (i + offb, 0, 0))
    out_shape = jax.ShapeDtypeStruct((_B, _S, _H), jnp.float32)
    if dest is None:
        return pl.pallas_call(
            _tc_ln_body,
            grid=(nb,),
            in_specs=base_specs,
            out_specs=out_spec,
            out_shape=out_shape,
        )(x, pos_table, tt3, t08, d8, gamma8, beta8)
    return pl.pallas_call(
        _tc_ln_part_body,
        grid=(nb,),
        in_specs=base_specs + [pl.BlockSpec(memory_space=pl.ANY)],
        out_specs=out_spec,
        out_shape=out_shape,
        input_output_aliases={7: 0},
    )(x, pos_table, tt3, t08, d8, gamma8, beta8, dest)


def kernel(input_ids, token_type_ids, word_table, pos_table, type_table,
           ln_gamma, ln_beta):
    ids = input_ids.reshape(-1).astype(jnp.int32)
    tt3 = token_type_ids.reshape(_B, 1, _S).astype(jnp.int32)
    t0 = type_table[0]
    d = type_table[1] - t0
    t08 = jnp.broadcast_to(t0[None, :], (8, _H))
    d8 = jnp.broadcast_to(d[None, :], (8, _H))
    gamma8 = jnp.broadcast_to(ln_gamma[None, :], (8, _H))
    beta8 = jnp.broadcast_to(ln_beta[None, :], (8, _H))

    nbp = _B // _NSPLIT
    ntok = nbp * _S
    sc_gather = _make_sc_gather(ntok)
    words = [sc_gather(lax.dynamic_slice_in_dim(ids, k * ntok, ntok),
                       word_table)
             for k in range(_NSPLIT)]
    out = None
    for k in range(_NSPLIT):
        out = _tc_ln_part(
            words[k].reshape(nbp, _S, _H), pos_table,
            lax.dynamic_slice_in_dim(tt3, k * nbp, nbp),
            t08, d8, gamma8, beta8, off=k * nbp, dest=out)
    mask = jnp.ones((_B, _S), dtype=jnp.int32)
    return (out, mask)


# speedup vs baseline: 1.8017x; 1.0123x over previous
import functools

import jax
import jax.numpy as jnp
from jax import lax
from jax.experimental import pallas as pl
from jax.experimental.pallas import tpu as pltpu
from jax.experimental.pallas import tpu_sc as plsc

_B, _S, _H = 64, 512, 768
_BS = _B * _S
_EPS = 1e-12
_CH = 64
_NSPLIT = 2


def _sc_info():
    info = plsc.get_sparse_core_info()
    return info.num_cores, info.num_subcores


def _sc_gather_body(nc, tpw, nchunk, ids_hbm, word_hbm, out_hbm,
                    idx0, idx1, buf0, buf1, gsem0, gsem1, osem0, osem1):
    wid = lax.axis_index("s") * nc + lax.axis_index("c")
    base = wid * tpw
    idxs = (idx0, idx1)
    bufs = (buf0, buf1)
    gsems = (gsem0, gsem1)
    osems = (osem0, osem1)

    def stage_and_fire(c):
        k = c & 1
        tok0 = base + c * _CH
        pltpu.sync_copy(ids_hbm.at[pl.ds(tok0, _CH)], idxs[k])
        return pltpu.async_copy(word_hbm.at[idxs[k]], bufs[k], gsems[k])

    gather = {0: stage_and_fire(0)}
    store = {}
    for c in range(nchunk):
        k = c & 1
        if c + 1 < nchunk:
            if c - 1 >= 0:
                store[c - 1].wait()
            gather[c + 1] = stage_and_fire(c + 1)
        gather[c].wait()
        tok0 = base + c * _CH
        store[c] = pltpu.async_copy(bufs[k], out_hbm.at[pl.ds(tok0, _CH)],
                                    osems[k])
    store[nchunk - 2].wait()
    store[nchunk - 1].wait()


def _make_sc_gather(ntok):
    nc, ns = _sc_info()
    nw = nc * ns
    tpw = ntok // nw
    mesh = plsc.VectorSubcoreMesh(core_axis_name="c", subcore_axis_name="s")
    return pl.kernel(
        functools.partial(_sc_gather_body, nc, tpw, tpw // _CH),
        mesh=mesh,
        out_type=jax.ShapeDtypeStruct((ntok, _H), jnp.float32),
        scratch_types=[
            pltpu.VMEM((_CH,), jnp.int32),
            pltpu.VMEM((_CH,), jnp.int32),
            pltpu.VMEM((_CH, _H), jnp.float32),
            pltpu.VMEM((_CH, _H), jnp.float32),
            pltpu.SemaphoreType.DMA,
            pltpu.SemaphoreType.DMA,
            pltpu.SemaphoreType.DMA,
            pltpu.SemaphoreType.DMA,
        ],
    )


_NBB = 4


def _tc_ln_body(x_ref, pos_ref, tt_ref, t0_ref, d_ref, g_ref, b_ref, o_ref):
    ttf = tt_ref[...].astype(jnp.float32)
    ttcol = jnp.swapaxes(ttf, 1, 2)
    x = x_ref[...] + (pos_ref[...] + t0_ref[0:1])[None] + ttcol * d_ref[0:1][None]
    mean = jnp.mean(x, axis=-1, keepdims=True)
    msq = jnp.mean(x * x, axis=-1, keepdims=True)
    inv = lax.rsqrt(msq - mean * mean + _EPS)
    o_ref[...] = (x - mean) * inv * g_ref[0:1][None] + b_ref[0:1][None]


def _tc_ln_part_body(x_ref, pos_ref, tt_ref, t0_ref, d_ref, g_ref, b_ref,
                     dest_ref, o_ref):
    del dest_ref
    _tc_ln_body(x_ref, pos_ref, tt_ref, t0_ref, d_ref, g_ref, b_ref, o_ref)


def _tc_ln_part(x, pos_table, tt3, t08, d8, gamma8, beta8, off, dest):
    nb = x.shape[0] // _NBB
    offb = off // _NBB
    base_specs = [
        pl.BlockSpec((_NBB, _S, _H), lambda i: (i, 0, 0)),
        pl.BlockSpec((_S, _H), lambda i: (0, 0)),
        pl.BlockSpec((_NBB, 1, _S), lambda i: (i, 0, 0)),
        pl.BlockSpec((8, _H), lambda i: (0, 0)),
        pl.BlockSpec((8, _H), lambda i: (0, 0)),
        pl.BlockSpec((8, _H), lambda i: (0, 0)),
        pl.BlockSpec((8, _H), lambda i: (0, 0)),
    ]
    out_spec = pl.BlockSpec((_NBB, _S, _H),
                            lambda i, offb=offb: (i + offb, 0, 0))
    out_shape = jax.ShapeDtypeStruct((_B, _S, _H), jnp.float32)
    if dest is None:
        return pl.pallas_call(
            _tc_ln_body,
            grid=(nb,),
            in_specs=base_specs,
            out_specs=out_spec,
            out_shape=out_shape,
        )(x, pos_table, tt3, t08, d8, gamma8, beta8)
    return pl.pallas_call(
        _tc_ln_part_body,
        grid=(nb,),
        in_specs=base_specs + [pl.BlockSpec(memory_space=pl.ANY)],
        out_specs=out_spec,
        out_shape=out_shape,
        input_output_aliases={7: 0},
    )(x, pos_table, tt3, t08, d8, gamma8, beta8, dest)


def kernel(input_ids, token_type_ids, word_table, pos_table, type_table,
           ln_gamma, ln_beta):
    ids = input_ids.reshape(-1).astype(jnp.int32)
    tt3 = token_type_ids.reshape(_B, 1, _S).astype(jnp.int32)
    t0 = type_table[0]
    d = type_table[1] - t0
    t08 = jnp.broadcast_to(t0[None, :], (8, _H))
    d8 = jnp.broadcast_to(d[None, :], (8, _H))
    gamma8 = jnp.broadcast_to(ln_gamma[None, :], (8, _H))
    beta8 = jnp.broadcast_to(ln_beta[None, :], (8, _H))

    nbp = _B // _NSPLIT
    ntok = nbp * _S
    sc_gather = _make_sc_gather(ntok)
    words = [sc_gather(lax.dynamic_slice_in_dim(ids, k * ntok, ntok),
                       word_table)
             for k in range(_NSPLIT)]
    out = None
    for k in range(_NSPLIT):
        out = _tc_ln_part(
            words[k].reshape(nbp, _S, _H), pos_table,
            lax.dynamic_slice_in_dim(tt3, k * nbp, nbp),
            t08, d8, gamma8, beta8, off=k * nbp, dest=out)
    mask = jnp.ones((_B, _S), dtype=jnp.int32)
    return (out, mask)
